# baseline (speedup 1.0000x reference)
import numpy as np
import jax
import jax.numpy as jnp
from jax import lax
from jax.experimental import pallas as pl
from jax.experimental.pallas import tpu as pltpu
from jax.experimental.pallas import tpu_sc as plsc

_KEYS = (
    (1832780943, 270669613),
    (64467757, 2916123636),
    (2465931498, 255383827),
    (3134548294, 894150801),
)

_T_MASK = 2516583
_T_ZERO = 6710887
_T_RAND = 838861

_ROT = ((13, 15, 26, 6), (17, 29, 16, 24))

_B, _T, _C = 64, 1024, 512
_ROWS = _B * _T
_BR = 512
_NB = _ROWS // _BR

_R_TC = 40960
_RB_TC = _R_TC // _BR
_R_SC = _ROWS - _R_TC

_N_TEC = 32
_TEC_ROWS = _R_SC // _N_TEC
_PIECE_ROWS = 64
_PIECES = _TEC_ROWS // _PIECE_ROWS
_PIECE_ELEMS = _PIECE_ROWS * _C
_VECS = _PIECE_ELEMS // 16


def _u32(v):
    return jnp.uint32(np.uint32(v))


def _cipher_xor(idx_u32, key):
    k0, k1 = key
    ks = (np.uint32(k0), np.uint32(k1),
          np.uint32(k0) ^ np.uint32(k1) ^ np.uint32(0x1BD11BDA))
    x1 = idx_u32 + _u32(ks[1])
    x0 = x1 + _u32(ks[0])
    first = True
    for r in range(5):
        for rot in _ROT[r % 2]:
            if first:
                first = False
            else:
                x0 = x0 + x1
            x1 = (x1 << _u32(rot)) | (x1 >> _u32(32 - rot))
            x1 = x1 ^ x0
        x0 = x0 + _u32(ks[(r + 1) % 3])
        x1 = x1 + _u32((int(ks[(r + 2) % 3]) + r + 1) & 0xFFFFFFFF)
    return x0 ^ x1


def _flags_from(idx_u32):
    m1 = _cipher_xor(idx_u32, _KEYS[0]) >> _u32(9)
    b1 = m1 < _u32(_T_MASK)
    m2 = _cipher_xor(idx_u32, _KEYS[1]) >> _u32(9)
    zero = (m2 < _u32(_T_ZERO)) & b1
    m3 = _cipher_xor(idx_u32, _KEYS[2]) >> _u32(9)
    rnd = (m3 < _u32(_T_RAND)) & b1 & jnp.logical_not(zero)
    return b1, zero, rnd


def _tc_pass1(spk_ref, flags_ref, max_ref):
    blk = pl.program_id(0)
    base = (blk * (_BR * _C)).astype(jnp.uint32)
    r = lax.broadcasted_iota(jnp.uint32, (_BR, _C), 0)
    c = lax.broadcasted_iota(jnp.uint32, (_BR, _C), 1)
    idx = base + r * _u32(_C) + c
    b1, zero, rnd = _flags_from(idx)
    flags_ref[...] = (b1.astype(jnp.int32) | (zero.astype(jnp.int32) << 1)
                      | (rnd.astype(jnp.int32) << 2))
    bm = jnp.max(jnp.where(zero, jnp.float32(0.0), spk_ref[...]))

    @pl.when(blk == 0)
    def _init():
        max_ref[0, 0] = bm

    @pl.when(blk > 0)
    def _acc():
        max_ref[0, 0] = jnp.maximum(max_ref[0, 0], bm)


def _sc_pass1(x_hbm, flags_hbm, maxp_hbm, spk_v, flg_v, acc_v):
    cid = lax.axis_index("c")
    sid = lax.axis_index("s")
    wid = sid * 2 + cid
    elem0 = (_R_TC + wid * _TEC_ROWS) * _C
    lbase = wid * (_TEC_ROWS * _C)
    lanes = lax.iota(jnp.int32, 16).astype(jnp.uint32)
    acc_v[...] = jnp.full((16,), jnp.float32(-3.0e38), dtype=jnp.float32)

    @pl.loop(0, _PIECES)
    def piece_body(p):
        estart = elem0 + p * _PIECE_ELEMS
        pltpu.sync_copy(x_hbm.at[pl.ds(estart, _PIECE_ELEMS)], spk_v)

        @pl.loop(0, _VECS, unroll=2)
        def vec_body(v):
            off = v * 16
            idx = (estart + off).astype(jnp.uint32) + lanes
            i_one = jnp.full((16,), 1, dtype=jnp.int32)
            i_zero = jnp.full((16,), 0, dtype=jnp.int32)
            m1 = _cipher_xor(idx, _KEYS[0]) >> _u32(9)
            f1 = jnp.where(m1 < _u32(_T_MASK), i_one, i_zero)
            m2 = _cipher_xor(idx, _KEYS[1]) >> _u32(9)
            f2 = jnp.where(m2 < _u32(_T_ZERO), i_one, i_zero)
            m3 = _cipher_xor(idx, _KEYS[2]) >> _u32(9)
            f3 = jnp.where(m3 < _u32(_T_RAND), i_one, i_zero)
            fz = f1 & f2
            fr = f3 & f1 & (i_one - fz)
            flg_v[pl.ds(off, 16)] = f1 | (fz << 1) | (fr << 2)
            spk = spk_v[pl.ds(off, 16)]
            acc_v[...] = jnp.maximum(
                acc_v[...], jnp.where(fz != 0, jnp.float32(0.0), spk))

        pltpu.sync_copy(
            flg_v, flags_hbm.at[pl.ds(lbase + p * _PIECE_ELEMS, _PIECE_ELEMS)])

    pltpu.sync_copy(acc_v, maxp_hbm.at[wid])


def _tc_pass2(max_ref, spk_ref, ftc_ref, fsc_ref, s_ref, mask_ref):
    blk = pl.program_id(0)
    base = (blk * (_BR * _C)).astype(jnp.uint32)
    r = lax.broadcasted_iota(jnp.uint32, (_BR, _C), 0)
    c = lax.broadcasted_iota(jnp.uint32, (_BR, _C), 1)
    idx = base + r * _u32(_C) + c
    f = jnp.where(blk < _RB_TC, ftc_ref[...], fsc_ref[...])
    mask_ref[...] = f & 1
    zero = (f & 2) != 0
    rnd = (f & 4) != 0
    m4 = _cipher_xor(idx, _KEYS[3]) >> _u32(9)
    u4 = m4.astype(jnp.float32) * jnp.float32(2.0 ** -23)
    rs = max_ref[0, 0] * u4
    s = jnp.where(zero, jnp.float32(0.0), spk_ref[...])
    s_ref[...] = jnp.where(rnd, rs, s)


def kernel(spikes):
    shp = spikes.shape
    x = spikes.reshape(_ROWS, _C)
    x1d = spikes.reshape(-1)

    flags_tc, mx_tc = pl.pallas_call(
        _tc_pass1,
        grid=(_RB_TC,),
        in_specs=[pl.BlockSpec((_BR, _C), lambda i: (i, 0))],
        out_specs=[
            pl.BlockSpec((_BR, _C), lambda i: (i, 0)),
            pl.BlockSpec(memory_space=pltpu.SMEM),
        ],
        out_shape=[
            jax.ShapeDtypeStruct((_R_TC, _C), jnp.int32),
            jax.ShapeDtypeStruct((1, 1), jnp.float32),
        ],
    )(x)

    sc_call = pl.kernel(
        _sc_pass1,
        out_type=[
            jax.ShapeDtypeStruct((_R_SC * _C,), jnp.int32),
            jax.ShapeDtypeStruct((_N_TEC, 16), jnp.float32),
        ],
        mesh=plsc.VectorSubcoreMesh(core_axis_name="c", subcore_axis_name="s"),
        scratch_types=[
            pltpu.VMEM((_PIECE_ELEMS,), jnp.float32),
            pltpu.VMEM((_PIECE_ELEMS,), jnp.int32),
            pltpu.VMEM((16,), jnp.float32),
        ],
    )
    flags_sc, maxp_sc = sc_call(x1d)

    mx = jnp.maximum(mx_tc[0, 0], jnp.max(maxp_sc)).reshape(1, 1)
    flags_sc = flags_sc.reshape(_R_SC, _C)

    s, mask = pl.pallas_call(
        _tc_pass2,
        grid=(_NB,),
        in_specs=[
            pl.BlockSpec(memory_space=pltpu.SMEM),
            pl.BlockSpec((_BR, _C), lambda i: (i, 0)),
            pl.BlockSpec((_BR, _C), lambda i: (jnp.minimum(i, _RB_TC - 1), 0)),
            pl.BlockSpec((_BR, _C), lambda i: (jnp.maximum(i - _RB_TC, 0), 0)),
        ],
        out_specs=[
            pl.BlockSpec((_BR, _C), lambda i: (i, 0)),
            pl.BlockSpec((_BR, _C), lambda i: (i, 0)),
        ],
        out_shape=[
            jax.ShapeDtypeStruct((_ROWS, _C), jnp.float32),
            jax.ShapeDtypeStruct((_ROWS, _C), jnp.int32),
        ],
    )(mx, x, flags_tc, flags_sc)

    return s.reshape(shp), mask.reshape(shp).astype(jnp.int64)


# speedup vs baseline: 1.0112x; 1.0112x over previous
import numpy as np
import jax
import jax.numpy as jnp
from jax import lax
from jax.experimental import pallas as pl
from jax.experimental.pallas import tpu as pltpu
from jax.experimental.pallas import tpu_sc as plsc

_KEYS = (
    (1832780943, 270669613),
    (64467757, 2916123636),
    (2465931498, 255383827),
    (3134548294, 894150801),
)

_T_MASK = 2516583
_T_ZERO = 6710887
_T_RAND = 838861

_ROT = ((13, 15, 26, 6), (17, 29, 16, 24))

_B, _T, _C = 64, 1024, 512
_ROWS = _B * _T
_BR = 512
_NB = _ROWS // _BR

_R_TC = 43008
_RB_TC = _R_TC // _BR
_R_SC = _ROWS - _R_TC

_N_TEC = 32
_TEC_ROWS = _R_SC // _N_TEC
_PIECE_ROWS = 64
_PIECES = _TEC_ROWS // _PIECE_ROWS
_PIECE_ELEMS = _PIECE_ROWS * _C
_VECS = _PIECE_ELEMS // 16


def _u32(v):
    return jnp.uint32(np.uint32(v))


def _cipher_xor(idx_u32, key):
    k0, k1 = key
    ks = (np.uint32(k0), np.uint32(k1),
          np.uint32(k0) ^ np.uint32(k1) ^ np.uint32(0x1BD11BDA))
    x1 = idx_u32 + _u32(ks[1])
    x0 = x1 + _u32(ks[0])
    first = True
    for r in range(5):
        for rot in _ROT[r % 2]:
            if first:
                first = False
            else:
                x0 = x0 + x1
            x1 = (x1 << _u32(rot)) | (x1 >> _u32(32 - rot))
            x1 = x1 ^ x0
        x0 = x0 + _u32(ks[(r + 1) % 3])
        x1 = x1 + _u32((int(ks[(r + 2) % 3]) + r + 1) & 0xFFFFFFFF)
    return x0 ^ x1


def _flags_from(idx_u32):
    b1 = _cipher_xor(idx_u32, _KEYS[0]) < _u32(_T_MASK << 9)
    zero = (_cipher_xor(idx_u32, _KEYS[1]) < _u32(_T_ZERO << 9)) & b1
    rnd = ((_cipher_xor(idx_u32, _KEYS[2]) < _u32(_T_RAND << 9))
           & b1 & jnp.logical_not(zero))
    return b1, zero, rnd


def _tc_pass1(spk_ref, flags_ref, max_ref):
    blk = pl.program_id(0)
    base = (blk * (_BR * _C)).astype(jnp.uint32)
    r = lax.broadcasted_iota(jnp.uint32, (_BR, _C), 0)
    c = lax.broadcasted_iota(jnp.uint32, (_BR, _C), 1)
    idx = base + r * _u32(_C) + c
    b1, zero, rnd = _flags_from(idx)
    flags_ref[...] = (b1.astype(jnp.int32) | (zero.astype(jnp.int32) << 1)
                      | (rnd.astype(jnp.int32) << 2))
    bm = jnp.max(jnp.where(zero, jnp.float32(0.0), spk_ref[...]))

    @pl.when(blk == 0)
    def _init():
        max_ref[0, 0] = bm

    @pl.when(blk > 0)
    def _acc():
        max_ref[0, 0] = jnp.maximum(max_ref[0, 0], bm)


def _sc_pass1(x_hbm, flags_hbm, maxp_hbm, spk_v, flg_v, acc_v):
    cid = lax.axis_index("c")
    sid = lax.axis_index("s")
    wid = sid * 2 + cid
    elem0 = (_R_TC + wid * _TEC_ROWS) * _C
    lbase = wid * (_TEC_ROWS * _C)
    lanes = lax.iota(jnp.int32, 16).astype(jnp.uint32)
    acc_v[...] = jnp.full((16,), jnp.float32(-3.0e38), dtype=jnp.float32)

    @pl.loop(0, _PIECES)
    def piece_body(p):
        estart = elem0 + p * _PIECE_ELEMS
        pltpu.sync_copy(x_hbm.at[pl.ds(estart, _PIECE_ELEMS)], spk_v)

        @pl.loop(0, _VECS, unroll=2)
        def vec_body(v):
            off = v * 16
            idx = (estart + off).astype(jnp.uint32) + lanes
            i_one = jnp.full((16,), 1, dtype=jnp.int32)
            i_zero = jnp.full((16,), 0, dtype=jnp.int32)
            f1 = jnp.where(_cipher_xor(idx, _KEYS[0]) < _u32(_T_MASK << 9),
                           i_one, i_zero)
            f2 = jnp.where(_cipher_xor(idx, _KEYS[1]) < _u32(_T_ZERO << 9),
                           i_one, i_zero)
            f3 = jnp.where(_cipher_xor(idx, _KEYS[2]) < _u32(_T_RAND << 9),
                           i_one, i_zero)
            fz = f1 & f2
            fr = f3 & f1 & (i_one - fz)
            flg_v[pl.ds(off, 16)] = f1 | (fz << 1) | (fr << 2)
            spk = spk_v[pl.ds(off, 16)]
            acc_v[...] = jnp.maximum(
                acc_v[...], jnp.where(fz != 0, jnp.float32(0.0), spk))

        pltpu.sync_copy(
            flg_v, flags_hbm.at[pl.ds(lbase + p * _PIECE_ELEMS, _PIECE_ELEMS)])

    pltpu.sync_copy(acc_v, maxp_hbm.at[wid])


def _tc_pass2(max_ref, spk_ref, ftc_ref, fsc_ref, s_ref, mask_ref):
    blk = pl.program_id(0)
    base = (blk * (_BR * _C)).astype(jnp.uint32)
    r = lax.broadcasted_iota(jnp.uint32, (_BR, _C), 0)
    c = lax.broadcasted_iota(jnp.uint32, (_BR, _C), 1)
    idx = base + r * _u32(_C) + c
    f = jnp.where(blk < _RB_TC, ftc_ref[...], fsc_ref[...])
    mask_ref[...] = f & 1
    zero = (f & 2) != 0
    rnd = (f & 4) != 0
    m4 = _cipher_xor(idx, _KEYS[3]) >> _u32(9)
    u4 = m4.astype(jnp.float32) * jnp.float32(2.0 ** -23)
    rs = max_ref[0, 0] * u4
    s = jnp.where(zero, jnp.float32(0.0), spk_ref[...])
    s_ref[...] = jnp.where(rnd, rs, s)


def kernel(spikes):
    shp = spikes.shape
    x = spikes.reshape(_ROWS, _C)
    x1d = spikes.reshape(-1)

    flags_tc, mx_tc = pl.pallas_call(
        _tc_pass1,
        grid=(_RB_TC,),
        in_specs=[pl.BlockSpec((_BR, _C), lambda i: (i, 0))],
        out_specs=[
            pl.BlockSpec((_BR, _C), lambda i: (i, 0)),
            pl.BlockSpec(memory_space=pltpu.SMEM),
        ],
        out_shape=[
            jax.ShapeDtypeStruct((_R_TC, _C), jnp.int32),
            jax.ShapeDtypeStruct((1, 1), jnp.float32),
        ],
    )(x)

    sc_call = pl.kernel(
        _sc_pass1,
        out_type=[
            jax.ShapeDtypeStruct((_R_SC * _C,), jnp.int32),
            jax.ShapeDtypeStruct((_N_TEC, 16), jnp.float32),
        ],
        mesh=plsc.VectorSubcoreMesh(core_axis_name="c", subcore_axis_name="s"),
        scratch_types=[
            pltpu.VMEM((_PIECE_ELEMS,), jnp.float32),
            pltpu.VMEM((_PIECE_ELEMS,), jnp.int32),
            pltpu.VMEM((16,), jnp.float32),
        ],
    )
    flags_sc, maxp_sc = sc_call(x1d)

    mx = jnp.maximum(mx_tc[0, 0], jnp.max(maxp_sc)).reshape(1, 1)
    flags_sc = flags_sc.reshape(_R_SC, _C)

    s, mask = pl.pallas_call(
        _tc_pass2,
        grid=(_NB,),
        in_specs=[
            pl.BlockSpec(memory_space=pltpu.SMEM),
            pl.BlockSpec((_BR, _C), lambda i: (i, 0)),
            pl.BlockSpec((_BR, _C), lambda i: (jnp.minimum(i, _RB_TC - 1), 0)),
            pl.BlockSpec((_BR, _C), lambda i: (jnp.maximum(i - _RB_TC, 0), 0)),
        ],
        out_specs=[
            pl.BlockSpec((_BR, _C), lambda i: (i, 0)),
            pl.BlockSpec((_BR, _C), lambda i: (i, 0)),
        ],
        out_shape=[
            jax.ShapeDtypeStruct((_ROWS, _C), jnp.float32),
            jax.ShapeDtypeStruct((_ROWS, _C), jnp.int32),
        ],
    )(mx, x, flags_tc, flags_sc)

    return s.reshape(shp), mask.reshape(shp).astype(jnp.int64)
